# baseline (speedup 1.0000x reference)
import functools

import jax
import jax.numpy as jnp
from jax import lax
from jax.experimental import pallas as pl
from jax.experimental.pallas import tpu as pltpu
from jax.experimental.pallas import tpu_sc as plsc

NC = 2
NS = 16
B = 80
SG = 25
NBUF = 4
ZC = 80


def _sc_aggregate(ego, rel, src3, dst3, w3, rsrc3, rdst3, rw3, n, d):
    nb = src3.shape[1]
    rows_per_tile = ((n + NS - 1) // NS + ZC - 1) // ZC * ZC
    npad = NS * rows_per_tile

    mesh = plsc.VectorSubcoreMesh(core_axis_name="c", subcore_axis_name="s",
                                  num_cores=NC, num_subcores=NS)

    @functools.partial(
        pl.kernel,
        out_type=jax.ShapeDtypeStruct((NC, npad, d), jnp.float32),
        mesh=mesh,
        compiler_params=pltpu.CompilerParams(use_tc_tiling_on_sc=False),
        scratch_types=[
            pltpu.VMEM_SHARED((npad, d), jnp.float32),
            pltpu.VMEM((SG, B), jnp.int32),
            pltpu.VMEM((SG, B), jnp.int32),
            pltpu.VMEM((SG, B), jnp.float32),
            [pltpu.VMEM((B, d), jnp.float32)] * NBUF,
            pltpu.VMEM((1, B), jnp.int32),
            [pltpu.SemaphoreType.DMA] * NBUF,
            [pltpu.SemaphoreType.DMA] * NBUF,
        ],
    )
    def sc_kernel(ego_h, rel_h, src_h, dst_h, w_h, rsrc_h, rdst_h, rw_h,
                  out_h, acc, srcs, dsts, ws, bufs, zidx, gsems, ssems):
        c = lax.axis_index("c")
        s = lax.axis_index("s")
        wid = c * NS + s

        zero = jnp.zeros((16,), jnp.float32)

        def zrow(i, _):
            for j in range(d // 16):
                sl = pl.ds(j * 16, 16)
                for r in range(NBUF):
                    bufs[r][i, sl] = zero
            return 0

        lax.fori_loop(0, B, zrow, 0)
        iota = lax.iota(jnp.int32, 16)
        for j in range(B // 16):
            zidx[0, pl.ds(j * 16, 16)] = iota + (j * 16)
        for ch in range(rows_per_tile // ZC):
            start = s * rows_per_tile + ch * ZC
            pltpu.sync_copy(bufs[0].at[pl.ds(0, ZC)], acc.at[pl.ds(start, ZC)])
        plsc.subcore_barrier()

        for r in range(NBUF):
            pltpu.async_copy(bufs[r], acc.at[zidx.at[0]], ssems[r], add=True)

        def wait_gather(table_h, r):
            pltpu.make_async_copy(table_h.at[srcs.at[0]], bufs[r],
                                  gsems[r]).wait()

        def wait_scatter(r):
            pltpu.make_async_copy(bufs[r], acc.at[dsts.at[0]],
                                  ssems[r]).wait()

        def scale_buf(buf, j, w_scale):
            def grp(g, _):
                wv = ws[j, pl.ds(g * 16, 16)] * w_scale
                for l in range(16):
                    w = wv[l]
                    e = g * 16 + l
                    for jj in range(d // 16):
                        sl = pl.ds(jj * 16, 16)
                        buf[e, sl] = buf[e, sl] * w
                return 0

            lax.fori_loop(0, B // 16, grp, 0)

        def do_pass(table_h, src_h3, dst_h3, w_h3, w_scale):
            def step(b, r, issue):
                wait_gather(table_h, r)
                if issue is not None:
                    p = (r + NBUF - 1) % NBUF

                    def do_issue():
                        wait_scatter(p)
                        pltpu.async_copy(table_h.at[srcs.at[b + NBUF - 1]],
                                         bufs[p], gsems[p])

                    if issue is True:
                        do_issue()
                    else:
                        pl.when(issue)(do_issue)
                scale_buf(bufs[r], b, w_scale)
                pltpu.async_copy(bufs[r], acc.at[dsts.at[b]], ssems[r],
                                 add=True)

            def stage_grp(sg, _):
                pltpu.sync_copy(src_h3.at[wid, pl.ds(sg * SG, SG)], srcs)
                pltpu.sync_copy(dst_h3.at[wid, pl.ds(sg * SG, SG)], dsts)
                pltpu.sync_copy(w_h3.at[wid, pl.ds(sg * SG, SG)], ws)

                for r in range(NBUF - 1):
                    wait_scatter(r)
                    pltpu.async_copy(table_h.at[srcs.at[r]], bufs[r],
                                     gsems[r])

                steps_per = SG // NBUF

                def rotate(t, _):
                    for r in range(NBUF):
                        thr = (SG - NBUF - r) // NBUF + 1
                        issue = True if thr >= steps_per else (t < thr)
                        step(NBUF * t + r, r, issue)
                    return 0

                lax.fori_loop(0, steps_per, rotate, 0)
                for j in range(NBUF * steps_per, SG):
                    step(j, j % NBUF, None)
                return 0

            lax.fori_loop(0, nb // SG, stage_grp, 0)

        do_pass(ego_h, src_h, dst_h, w_h, 1.0)
        do_pass(rel_h, rsrc_h, rdst_h, rw_h, 0.1)

        for r in range(NBUF):
            wait_scatter(r)
        plsc.subcore_barrier()
        for ch in range(rows_per_tile // ZC):
            start = s * rows_per_tile + ch * ZC
            pltpu.sync_copy(acc.at[pl.ds(start, ZC)],
                            out_h.at[c, pl.ds(start, ZC), :])

    return sc_kernel(ego, rel, src3, dst3, w3, rsrc3, rdst3, rw3)


def _tc_finish(ego, parts, W_w, b2, n, d):
    rows_blk = 1000
    grid = (n // rows_blk,)

    def tc_body(ego_ref, parts_ref, w_ref, b_ref, out_ref):
        x = ego_ref[...] + parts_ref[0] + parts_ref[1]
        pre = lax.dot_general(x, w_ref[...], (((1,), (1,)), ((), ())),
                              preferred_element_type=jnp.float32)
        pre = pre + b_ref[...]
        out_ref[...] = jnp.where(pre >= 0, pre, pre * 0.01)

    return pl.pallas_call(
        tc_body,
        grid=grid,
        in_specs=[
            pl.BlockSpec((rows_blk, d), lambda i: (i, 0)),
            pl.BlockSpec((2, rows_blk, d), lambda i: (0, i, 0)),
            pl.BlockSpec((d, d), lambda i: (0, 0)),
            pl.BlockSpec((1, d), lambda i: (0, 0)),
        ],
        out_specs=pl.BlockSpec((rows_blk, d), lambda i: (i, 0)),
        out_shape=jax.ShapeDtypeStruct((n, d), jnp.float32),
    )(ego, parts, W_w, b2)


def _pad_edges(src, dst, w, idx_n, e_pad):
    e = w.shape[0]
    if e_pad == e:
        return src, dst, w
    extra = e_pad - e
    fill = (jnp.arange(extra, dtype=jnp.int32) * 97) % jnp.int32(idx_n)
    src = jnp.concatenate([src, fill])
    dst = jnp.concatenate([dst, fill])
    w = jnp.concatenate([w, jnp.zeros((extra,), w.dtype)])
    return src, dst, w


def kernel(ego_embeddings, rel_embeddings, edge_index, edge_weight,
           rel_edge_index, rel_edge_weight, W_w, W_b):
    n, d = ego_embeddings.shape
    r_n = rel_embeddings.shape[0]
    e = edge_weight.shape[0]
    nw = NC * NS
    per_tile = (e + nw - 1) // nw
    per_tile = (per_tile + B * SG - 1) // (B * SG) * (B * SG)
    e_pad = per_tile * nw
    nb = per_tile // B

    src, dst, w = _pad_edges(edge_index[1], edge_index[0], edge_weight,
                             n, e_pad)
    rsrc, rdst, rw = _pad_edges(rel_edge_index[1], rel_edge_index[0],
                                rel_edge_weight, r_n, e_pad)
    shape3 = (nw, nb, B)
    parts = _sc_aggregate(ego_embeddings, rel_embeddings,
                          src.reshape(shape3), dst.reshape(shape3),
                          w.reshape(shape3), rsrc.reshape(shape3),
                          rdst.reshape(shape3), rw.reshape(shape3), n, d)
    return _tc_finish(ego_embeddings, parts, W_w,
                      W_b.reshape(1, d), n, d)


# speedup vs baseline: 1.0011x; 1.0011x over previous
import functools

import jax
import jax.numpy as jnp
from jax import lax
from jax.experimental import pallas as pl
from jax.experimental.pallas import tpu as pltpu
from jax.experimental.pallas import tpu_sc as plsc

NC = 2
NS = 16
B = 80
SG = 25
NBUF = 3
ZC = 80


def _sc_aggregate(ego, rel, src3, dst3, w3, rsrc3, rdst3, rw3, n, d):
    nb = src3.shape[1]
    rows_per_tile = ((n + NS - 1) // NS + ZC - 1) // ZC * ZC
    npad = NS * rows_per_tile

    mesh = plsc.VectorSubcoreMesh(core_axis_name="c", subcore_axis_name="s",
                                  num_cores=NC, num_subcores=NS)

    @functools.partial(
        pl.kernel,
        out_type=jax.ShapeDtypeStruct((NC, npad, d), jnp.float32),
        mesh=mesh,
        compiler_params=pltpu.CompilerParams(use_tc_tiling_on_sc=False),
        scratch_types=[
            pltpu.VMEM_SHARED((npad, d), jnp.float32),
            pltpu.VMEM((SG, B), jnp.int32),
            pltpu.VMEM((SG, B), jnp.int32),
            pltpu.VMEM((SG, B), jnp.float32),
            [pltpu.VMEM((B, d), jnp.float32)] * NBUF,
            pltpu.VMEM((1, B), jnp.int32),
            [pltpu.SemaphoreType.DMA] * NBUF,
            [pltpu.SemaphoreType.DMA] * NBUF,
        ],
    )
    def sc_kernel(ego_h, rel_h, src_h, dst_h, w_h, rsrc_h, rdst_h, rw_h,
                  out_h, acc, srcs, dsts, ws, bufs, zidx, gsems, ssems):
        c = lax.axis_index("c")
        s = lax.axis_index("s")
        wid = c * NS + s

        zero = jnp.zeros((16,), jnp.float32)

        def zrow(i, _):
            for j in range(d // 16):
                sl = pl.ds(j * 16, 16)
                for r in range(NBUF):
                    bufs[r][i, sl] = zero
            return 0

        lax.fori_loop(0, B, zrow, 0)
        iota = lax.iota(jnp.int32, 16)
        for j in range(B // 16):
            zidx[0, pl.ds(j * 16, 16)] = iota + (j * 16)
        for ch in range(rows_per_tile // ZC):
            start = s * rows_per_tile + ch * ZC
            pltpu.sync_copy(bufs[0].at[pl.ds(0, ZC)], acc.at[pl.ds(start, ZC)])
        plsc.subcore_barrier()

        for r in range(NBUF):
            pltpu.async_copy(bufs[r], acc.at[zidx.at[0]], ssems[r], add=True)

        def wait_gather(table_h, r):
            pltpu.make_async_copy(table_h.at[srcs.at[0]], bufs[r],
                                  gsems[r]).wait()

        def wait_scatter(r):
            pltpu.make_async_copy(bufs[r], acc.at[dsts.at[0]],
                                  ssems[r]).wait()

        def scale_buf(buf, j, w_scale):
            def grp(g, _):
                wv = ws[j, pl.ds(g * 16, 16)] * w_scale
                for l in range(16):
                    w = wv[l]
                    e = g * 16 + l
                    for jj in range(d // 16):
                        sl = pl.ds(jj * 16, 16)
                        buf[e, sl] = buf[e, sl] * w
                return 0

            lax.fori_loop(0, B // 16, grp, 0)

        def do_pass(table_h, src_h3, dst_h3, w_h3, w_scale):
            def step(b, r, issue):
                wait_gather(table_h, r)
                if issue is not None:
                    p = (r + NBUF - 1) % NBUF

                    def do_issue():
                        wait_scatter(p)
                        pltpu.async_copy(table_h.at[srcs.at[b + NBUF - 1]],
                                         bufs[p], gsems[p])

                    if issue is True:
                        do_issue()
                    else:
                        pl.when(issue)(do_issue)
                scale_buf(bufs[r], b, w_scale)
                pltpu.async_copy(bufs[r], acc.at[dsts.at[b]], ssems[r],
                                 add=True)

            def stage_grp(sg, _):
                pltpu.sync_copy(src_h3.at[wid, pl.ds(sg * SG, SG)], srcs)
                pltpu.sync_copy(dst_h3.at[wid, pl.ds(sg * SG, SG)], dsts)
                pltpu.sync_copy(w_h3.at[wid, pl.ds(sg * SG, SG)], ws)

                for r in range(NBUF - 1):
                    wait_scatter(r)
                    pltpu.async_copy(table_h.at[srcs.at[r]], bufs[r],
                                     gsems[r])

                steps_per = SG // NBUF

                def rotate(t, _):
                    for r in range(NBUF):
                        thr = (SG - NBUF - r) // NBUF + 1
                        issue = True if thr >= steps_per else (t < thr)
                        step(NBUF * t + r, r, issue)
                    return 0

                lax.fori_loop(0, steps_per, rotate, 0)
                for j in range(NBUF * steps_per, SG):
                    step(j, j % NBUF, None)
                return 0

            lax.fori_loop(0, nb // SG, stage_grp, 0)

        do_pass(ego_h, src_h, dst_h, w_h, 1.0)
        do_pass(rel_h, rsrc_h, rdst_h, rw_h, 0.1)

        for r in range(NBUF):
            wait_scatter(r)
        plsc.subcore_barrier()
        for ch in range(rows_per_tile // ZC):
            start = s * rows_per_tile + ch * ZC
            pltpu.sync_copy(acc.at[pl.ds(start, ZC)],
                            out_h.at[c, pl.ds(start, ZC), :])

    return sc_kernel(ego, rel, src3, dst3, w3, rsrc3, rdst3, rw3)


def _tc_finish(ego, parts, W_w, b2, n, d):
    rows_blk = 1000
    grid = (n // rows_blk,)

    def tc_body(ego_ref, parts_ref, w_ref, b_ref, out_ref):
        x = ego_ref[...] + parts_ref[0] + parts_ref[1]
        pre = lax.dot_general(x, w_ref[...], (((1,), (1,)), ((), ())),
                              preferred_element_type=jnp.float32)
        pre = pre + b_ref[...]
        out_ref[...] = jnp.where(pre >= 0, pre, pre * 0.01)

    return pl.pallas_call(
        tc_body,
        grid=grid,
        in_specs=[
            pl.BlockSpec((rows_blk, d), lambda i: (i, 0)),
            pl.BlockSpec((2, rows_blk, d), lambda i: (0, i, 0)),
            pl.BlockSpec((d, d), lambda i: (0, 0)),
            pl.BlockSpec((1, d), lambda i: (0, 0)),
        ],
        out_specs=pl.BlockSpec((rows_blk, d), lambda i: (i, 0)),
        out_shape=jax.ShapeDtypeStruct((n, d), jnp.float32),
    )(ego, parts, W_w, b2)


def _pad_edges(src, dst, w, idx_n, e_pad):
    e = w.shape[0]
    if e_pad == e:
        return src, dst, w
    extra = e_pad - e
    fill = (jnp.arange(extra, dtype=jnp.int32) * 97) % jnp.int32(idx_n)
    src = jnp.concatenate([src, fill])
    dst = jnp.concatenate([dst, fill])
    w = jnp.concatenate([w, jnp.zeros((extra,), w.dtype)])
    return src, dst, w


def kernel(ego_embeddings, rel_embeddings, edge_index, edge_weight,
           rel_edge_index, rel_edge_weight, W_w, W_b):
    n, d = ego_embeddings.shape
    r_n = rel_embeddings.shape[0]
    e = edge_weight.shape[0]
    nw = NC * NS
    per_tile = (e + nw - 1) // nw
    per_tile = (per_tile + B * SG - 1) // (B * SG) * (B * SG)
    e_pad = per_tile * nw
    nb = per_tile // B

    src, dst, w = _pad_edges(edge_index[1], edge_index[0], edge_weight,
                             n, e_pad)
    rsrc, rdst, rw = _pad_edges(rel_edge_index[1], rel_edge_index[0],
                                rel_edge_weight, r_n, e_pad)
    shape3 = (nw, nb, B)
    parts = _sc_aggregate(ego_embeddings, rel_embeddings,
                          src.reshape(shape3), dst.reshape(shape3),
                          w.reshape(shape3), rsrc.reshape(shape3),
                          rdst.reshape(shape3), rw.reshape(shape3), n, d)
    return _tc_finish(ego_embeddings, parts, W_w,
                      W_b.reshape(1, d), n, d)
